# Initial kernel scaffold; baseline (speedup 1.0000x reference)
import jax
import jax.numpy as jnp
from jax.experimental import pallas as pl


def kernel(character_labels, char_emb, not_char_emb, sep_emb):
    raise NotImplementedError("write your pallas kernel here")


# speedup vs baseline: 16.6216x; 16.6216x over previous
import jax
import jax.numpy as jnp
from jax.experimental import pallas as pl
from jax.experimental.pallas import tpu as pltpu

N_CHARS = 9
DIM = 1024
MULT_DIM = 8
ROWS = N_CHARS * MULT_DIM
BB = 8


def _body(bits_ref, ce_ref, ne_ref, sep_ref, out_ref):
    i = pl.program_id(0)
    ce = ce_ref[...].reshape(ROWS, DIM)
    ne = ne_ref[...].reshape(ROWS, DIM)
    sep = sep_ref[...]
    sep72 = jnp.broadcast_to(sep[:, None, :], (N_CHARS, MULT_DIM, DIM)).reshape(ROWS, DIM)
    cemb = ce + sep72
    nemb = ne + sep72
    for r in range(BB):
        bits = bits_ref[i * BB + r]
        for c in range(N_CHARS):
            sel = ((bits >> c) & 1) == 1
            lo = c * MULT_DIM
            out_ref[r, lo:lo + MULT_DIM, :] = jnp.where(
                sel, cemb[lo:lo + MULT_DIM, :], nemb[lo:lo + MULT_DIM, :])


def kernel(character_labels, char_emb, not_char_emb, sep_emb):
    b = character_labels.shape[0]
    bits = jnp.sum(
        character_labels.astype(jnp.int32) << jnp.arange(N_CHARS, dtype=jnp.int32)[None, :],
        axis=1, dtype=jnp.int32)
    grid = (b // BB,)
    out = pl.pallas_call(
        _body,
        grid=grid,
        in_specs=[
            pl.BlockSpec(memory_space=pltpu.SMEM),
            pl.BlockSpec((N_CHARS, DIM * MULT_DIM), lambda i: (0, 0)),
            pl.BlockSpec((N_CHARS, DIM * MULT_DIM), lambda i: (0, 0)),
            pl.BlockSpec((N_CHARS, DIM), lambda i: (0, 0)),
        ],
        out_specs=pl.BlockSpec((BB, ROWS, DIM), lambda i: (i, 0, 0)),
        out_shape=jax.ShapeDtypeStruct((b, ROWS, DIM), jnp.float32),
    )(bits, char_emb, not_char_emb, sep_emb)
    return out
